# baseline (speedup 1.0000x reference)
import functools

import jax
import jax.numpy as jnp
from jax import lax
from jax.experimental import pallas as pl
from jax.experimental.pallas import tpu as pltpu
from jax.experimental.pallas import tpu_sc as plsc

B = 16384
NF = 64
D_IMG = 512
NU = 1000000
NC = 2
NS = 16
NW = NC * NS
EPW = B // NW
UPW = 31232
CW = 1280
NCH = 24
TAIL_LO = NW * UPW
OUT_ROWS = B + 16

_sc_mesh = plsc.VectorSubcoreMesh(core_axis_name="c", subcore_axis_name="s")


@functools.partial(
    pl.kernel,
    mesh=_sc_mesh,
    out_type=[
        jax.ShapeDtypeStruct((OUT_ROWS, 128), jnp.float32),
        jax.ShapeDtypeStruct((B,), jnp.float32),
    ],
    scratch_types=[
        pltpu.VMEM((EPW,), jnp.int32),
        pltpu.VMEM((EPW,), jnp.int32),
        pltpu.VMEM((EPW,), jnp.float32),
        pltpu.VMEM((EPW,), jnp.float32),
        pltpu.VMEM((EPW,), jnp.float32),
        pltpu.VMEM((1024,), jnp.int32),
        pltpu.VMEM((B,), jnp.int32),
        pltpu.VMEM((B,), jnp.int32),
        pltpu.VMEM((NF, CW), jnp.float32),
        pltpu.VMEM((NF, 64), jnp.float32),
        pltpu.VMEM((16, 128), jnp.float32),
        pltpu.VMEM((16,), jnp.int32),
        pltpu.VMEM((16,), jnp.int32),
        pltpu.SemaphoreType.DMA,
        pltpu.SemaphoreType.DMA,
    ],
    compiler_params=pltpu.CompilerParams(needs_layout_passes=False),
)
def _sc_gather(user_hbm, item_hbm, ufT_hbm, tailT_hbm, ub_hbm, ib_hbm,
               out_hbm, bias_hbm,
               uidx_v, iidx_v, ubg_v, ibg_v, bias_v, uv_buf,
               el_v, ul_v, strip_v, tail_v, pack_v, tu_v, te_v, sem, sem2):
    wid = lax.axis_index("s") * NC + lax.axis_index("c")
    ebase = wid * EPW

    pltpu.sync_copy(user_hbm.at[pl.ds(ebase, EPW)], uidx_v)
    pltpu.sync_copy(item_hbm.at[pl.ds(ebase, EPW)], iidx_v)
    copies = []
    for j in range(EPW // 128):
        sl = pl.ds(j * 128, 128)
        copies.append(pltpu.async_copy(ub_hbm.at[uidx_v.at[sl]],
                                       ubg_v.at[sl], sem))
        copies.append(pltpu.async_copy(ib_hbm.at[iidx_v.at[sl]],
                                       ibg_v.at[sl], sem))
    for c in copies:
        c.wait()
    for j in range(EPW // 16):
        sl = pl.ds(j * 16, 16)
        bias_v[sl] = ubg_v[sl] + ibg_v[sl]
    pltpu.sync_copy(bias_v, bias_hbm.at[pl.ds(ebase, EPW)])

    lo = wid * UPW
    hi = lo + UPW
    is_w0 = wid == 0
    iota16 = lax.iota(jnp.int32, 16)

    def scan_tile(t, cnt):
        def scan_blk(b16, cnt):
            u16 = uv_buf[pl.ds(b16 * 16, 16)]
            m = jnp.logical_and(u16 >= lo, u16 < hi)
            m = jnp.logical_or(m, jnp.logical_and(is_w0, u16 >= TAIL_LO))
            plsc.store_compressed(el_v.at[pl.ds(cnt, 16)],
                                  t * 1024 + b16 * 16 + iota16, mask=m)
            plsc.store_compressed(ul_v.at[pl.ds(cnt, 16)], u16, mask=m)
            return cnt + jnp.sum(m.astype(jnp.int32))
        return lax.fori_loop(0, 64, scan_blk, cnt)

    def bin_tile(t, cnt):
        pltpu.sync_copy(user_hbm.at[pl.ds(t * 1024, 1024)], uv_buf)
        return scan_tile(t, cnt)

    cnt = lax.fori_loop(0, 16, bin_tile, jnp.int32(0))
    nblk = (cnt + 15) >> 4
    dummy = jnp.int32(B + 8)

    def extract_blocks(buf, clo, width):
        @pl.loop(0, nblk)
        def blk_body(j):
            u16 = ul_v[pl.ds(j * 16, 16)]
            e16 = el_v[pl.ds(j * 16, 16)]
            valid = iota16 < (cnt - j * 16)
            m = jnp.logical_and(valid,
                                jnp.logical_and(u16 >= clo, u16 < clo + width))
            c16 = jnp.sum(m.astype(jnp.int32))

            @pl.when(c16 > 0)
            def _():
                plsc.store_compressed(tu_v.at[pl.ds(0, 16)], u16, mask=m)
                plsc.store_compressed(te_v.at[pl.ds(0, 16)], e16, mask=m)
                uu = tu_v[pl.ds(0, 16)]
                ee = te_v[pl.ds(0, 16)]
                lanes = jnp.clip(uu - clo, 0, width - 1)
                mk = iota16 < c16
                for k in range(NF):
                    kv = jnp.full((16,), k, jnp.int32)
                    v = plsc.load_gather(buf, [kv, lanes])
                    plsc.store_scatter(pack_v, [iota16, kv], v)
                rows = jnp.where(mk, ee, dummy)
                pltpu.async_copy(pack_v, out_hbm.at[rows], sem2).wait()

    def process_chunk(clo, width):
        strips = [
            pltpu.async_copy(
                ufT_hbm.at[pl.ds(kg * 8, 8), pl.ds(clo, width)],
                strip_v.at[pl.ds(kg * 8, 8), pl.ds(0, width)], sem2)
            for kg in range(8)
        ]
        for s in strips:
            s.wait()
        extract_blocks(strip_v, clo, width)

    @pl.loop(0, NCH)
    def chunk_body(ch):
        process_chunk(lo + ch * CW, CW)

    process_chunk(lo + NCH * CW, 512)

    @pl.when(is_w0)
    def _tail():
        process_chunk(jnp.int32(TAIL_LO), 512)
        pltpu.sync_copy(tailT_hbm, tail_v)
        extract_blocks(tail_v, jnp.int32(TAIL_LO + 512), 64)


BLK = 1024

_tc_matmul = pl.pallas_call(
    lambda img_ref, w_ref, b_ref, o_ref: o_ref.__setitem__(
        ..., jnp.dot(img_ref[...], w_ref[...],
                     preferred_element_type=jnp.float32) + b_ref[...]),
    grid=(B // BLK,),
    in_specs=[
        pl.BlockSpec((BLK, D_IMG), lambda i: (i, 0)),
        pl.BlockSpec((D_IMG, NF), lambda i: (0, 0)),
        pl.BlockSpec((1, NF), lambda i: (0, 0)),
    ],
    out_specs=pl.BlockSpec((BLK, NF), lambda i: (i, 0)),
    out_shape=jax.ShapeDtypeStruct((B, NF), jnp.float32),
)


def _tc_comb_body(imf_ref, pk_ref, bias_ref, out_ref):
    ue = pk_ref[:, :NF]
    out_ref[...] = jnp.sum(imf_ref[...] * ue, axis=1) + bias_ref[...]


_tc_combine = pl.pallas_call(
    _tc_comb_body,
    grid=(B // BLK,),
    in_specs=[
        pl.BlockSpec((BLK, NF), lambda i: (i, 0)),
        pl.BlockSpec((BLK, 128), lambda i: (i, 0)),
        pl.BlockSpec((BLK,), lambda i: (i,)),
    ],
    out_specs=pl.BlockSpec((BLK,), lambda i: (i,)),
    out_shape=jax.ShapeDtypeStruct((B,), jnp.float32),
)


def kernel(image, user, item, user_factors, user_biases, item_biases,
           W_img, b_img):
    user = user.astype(jnp.int32)
    item = item.astype(jnp.int32)
    ufT = user_factors.T
    tailT = ufT[:, TAIL_LO + 512:]
    ub_flat = user_biases.reshape(-1)
    ib_flat = item_biases.reshape(-1)
    packed, bias = _sc_gather(user, item, ufT, tailT, ub_flat, ib_flat)
    imf = _tc_matmul(image, W_img, b_img.reshape(1, NF))
    return _tc_combine(imf, packed, bias)


# speedup vs baseline: 23.6518x; 23.6518x over previous
import functools

import jax
import jax.numpy as jnp
from jax import lax
from jax.experimental import pallas as pl
from jax.experimental.pallas import tpu as pltpu
from jax.experimental.pallas import tpu_sc as plsc

B = 16384
NF = 64
D_IMG = 512
NU = 1000000
NC = 2
NS = 16
NW = NC * NS
EPW = B // NW
UPW = 31232
CW = 1280
NCH = 24
TAIL_LO = NW * UPW
OUT_ROWS = B + NW

_sc_mesh = plsc.VectorSubcoreMesh(core_axis_name="c", subcore_axis_name="s")


@functools.partial(
    pl.kernel,
    mesh=_sc_mesh,
    out_type=[
        jax.ShapeDtypeStruct((OUT_ROWS, 128), jnp.float32),
        jax.ShapeDtypeStruct((B,), jnp.float32),
    ],
    scratch_types=[
        pltpu.VMEM((EPW,), jnp.int32),
        pltpu.VMEM((EPW,), jnp.int32),
        pltpu.VMEM((EPW,), jnp.float32),
        pltpu.VMEM((EPW,), jnp.float32),
        pltpu.VMEM((EPW,), jnp.float32),
        pltpu.VMEM((1024,), jnp.int32),
        pltpu.VMEM((B,), jnp.int32),
        pltpu.VMEM((B,), jnp.int32),
        pltpu.VMEM((NF, CW), jnp.float32),
        pltpu.VMEM((NF, 64), jnp.float32),
        pltpu.VMEM((16, 128), jnp.float32),
        pltpu.VMEM((32,), jnp.int32),
        pltpu.VMEM((32,), jnp.int32),
        pltpu.SemaphoreType.DMA,
        pltpu.SemaphoreType.DMA,
    ],
    compiler_params=pltpu.CompilerParams(needs_layout_passes=False),
)
def _sc_gather(user_hbm, item_hbm, ufT_hbm, tailT_hbm, ub_hbm, ib_hbm,
               out_hbm, bias_hbm,
               uidx_v, iidx_v, ubg_v, ibg_v, bias_v, uv_buf,
               el_v, ul_v, strip_v, tail_v, pack_v, tu_v, te_v, sem, sem2):
    wid = lax.axis_index("s") * NC + lax.axis_index("c")
    ebase = wid * EPW

    pltpu.sync_copy(user_hbm.at[pl.ds(ebase, EPW)], uidx_v)
    pltpu.sync_copy(item_hbm.at[pl.ds(ebase, EPW)], iidx_v)
    copies = []
    for j in range(EPW // 128):
        sl = pl.ds(j * 128, 128)
        copies.append(pltpu.async_copy(ub_hbm.at[uidx_v.at[sl]],
                                       ubg_v.at[sl], sem))
        copies.append(pltpu.async_copy(ib_hbm.at[iidx_v.at[sl]],
                                       ibg_v.at[sl], sem))
    for c in copies:
        c.wait()
    for j in range(EPW // 16):
        sl = pl.ds(j * 16, 16)
        bias_v[sl] = ubg_v[sl] + ibg_v[sl]
    pltpu.sync_copy(bias_v, bias_hbm.at[pl.ds(ebase, EPW)])

    lo = wid * UPW
    hi = lo + UPW
    is_w0 = wid == 0
    iota16 = lax.iota(jnp.int32, 16)

    def scan_tile(t, cnt):
        def scan_blk(b16, cnt):
            u16 = uv_buf[pl.ds(b16 * 16, 16)]
            m = jnp.logical_and(u16 >= lo, u16 < hi)
            m = jnp.logical_or(m, jnp.logical_and(is_w0, u16 >= TAIL_LO))
            plsc.store_compressed(el_v.at[pl.ds(cnt, 16)],
                                  t * 1024 + b16 * 16 + iota16, mask=m)
            plsc.store_compressed(ul_v.at[pl.ds(cnt, 16)], u16, mask=m)
            return cnt + jnp.sum(m.astype(jnp.int32))
        return lax.fori_loop(0, 64, scan_blk, cnt)

    def bin_tile(t, cnt):
        pltpu.sync_copy(user_hbm.at[pl.ds(t * 1024, 1024)], uv_buf)
        return scan_tile(t, cnt)

    cnt = lax.fori_loop(0, 16, bin_tile, jnp.int32(0))
    nblk = (cnt + 15) >> 4
    dummy = B + wid

    def flush(buf, clo, width, nvalid):
        uu = tu_v[pl.ds(0, 16)]
        ee = te_v[pl.ds(0, 16)]
        lanes = jnp.clip(uu - clo, 0, width - 1)
        mk = iota16 < nvalid
        for k in range(NF):
            kv = jnp.full((16,), k, jnp.int32)
            v = plsc.load_gather(buf, [kv, lanes])
            plsc.store_scatter(pack_v, [iota16, kv], v)
        rows = jnp.where(mk, ee, dummy)
        pltpu.async_copy(pack_v, out_hbm.at[rows], sem2).wait()

    def extract_blocks(buf, clo, width):
        def blk_body(j, pc):
            u16 = ul_v[pl.ds(j * 16, 16)]
            e16 = el_v[pl.ds(j * 16, 16)]
            valid = iota16 < (cnt - j * 16)
            m = jnp.logical_and(valid,
                                jnp.logical_and(u16 >= clo, u16 < clo + width))
            c16 = jnp.sum(m.astype(jnp.int32))

            @pl.when(c16 > 0)
            def _():
                plsc.store_compressed(tu_v.at[pl.ds(pc, 16)], u16, mask=m)
                plsc.store_compressed(te_v.at[pl.ds(pc, 16)], e16, mask=m)

            pc2 = pc + c16

            @pl.when(pc2 >= 16)
            def _():
                flush(buf, clo, width, jnp.int32(16))
                tu_v[pl.ds(0, 16)] = tu_v[pl.ds(16, 16)]
                te_v[pl.ds(0, 16)] = te_v[pl.ds(16, 16)]

            return jnp.where(pc2 >= 16, pc2 - 16, pc2)

        pc = lax.fori_loop(0, nblk, blk_body, jnp.int32(0))

        @pl.when(pc > 0)
        def _():
            flush(buf, clo, width, pc)

    def process_chunk(clo, width):
        strips = [
            pltpu.async_copy(
                ufT_hbm.at[pl.ds(kg * 8, 8), pl.ds(clo, width)],
                strip_v.at[pl.ds(kg * 8, 8), pl.ds(0, width)], sem2)
            for kg in range(8)
        ]
        for s in strips:
            s.wait()
        extract_blocks(strip_v, clo, width)

    @pl.loop(0, NCH)
    def chunk_body(ch):
        process_chunk(lo + ch * CW, CW)

    process_chunk(lo + NCH * CW, 512)

    @pl.when(is_w0)
    def _tail():
        process_chunk(jnp.int32(TAIL_LO), 512)
        pltpu.sync_copy(tailT_hbm, tail_v)
        extract_blocks(tail_v, jnp.int32(TAIL_LO + 512), 64)


BLK = 1024

_tc_matmul = pl.pallas_call(
    lambda img_ref, w_ref, b_ref, o_ref: o_ref.__setitem__(
        ..., jnp.dot(img_ref[...], w_ref[...],
                     preferred_element_type=jnp.float32) + b_ref[...]),
    grid=(B // BLK,),
    in_specs=[
        pl.BlockSpec((BLK, D_IMG), lambda i: (i, 0)),
        pl.BlockSpec((D_IMG, NF), lambda i: (0, 0)),
        pl.BlockSpec((1, NF), lambda i: (0, 0)),
    ],
    out_specs=pl.BlockSpec((BLK, NF), lambda i: (i, 0)),
    out_shape=jax.ShapeDtypeStruct((B, NF), jnp.float32),
)


def _tc_comb_body(imf_ref, pk_ref, bias_ref, out_ref):
    ue = pk_ref[:, :NF]
    out_ref[...] = jnp.sum(imf_ref[...] * ue, axis=1) + bias_ref[...]


_tc_combine = pl.pallas_call(
    _tc_comb_body,
    grid=(B // BLK,),
    in_specs=[
        pl.BlockSpec((BLK, NF), lambda i: (i, 0)),
        pl.BlockSpec((BLK, 128), lambda i: (i, 0)),
        pl.BlockSpec((BLK,), lambda i: (i,)),
    ],
    out_specs=pl.BlockSpec((BLK,), lambda i: (i,)),
    out_shape=jax.ShapeDtypeStruct((B,), jnp.float32),
)


def kernel(image, user, item, user_factors, user_biases, item_biases,
           W_img, b_img):
    user = user.astype(jnp.int32)
    item = item.astype(jnp.int32)
    ufT = user_factors.T
    tailT = ufT[:, TAIL_LO + 512:]
    ub_flat = user_biases.reshape(-1)
    ib_flat = item_biases.reshape(-1)
    packed, bias = _sc_gather(user, item, ufT, tailT, ub_flat, ib_flat)
    imf = _tc_matmul(image, W_img, b_img.reshape(1, NF))
    return _tc_combine(imf, packed, bias)


# speedup vs baseline: 32.1183x; 1.3580x over previous
import functools

import jax
import jax.numpy as jnp
from jax import lax
from jax.experimental import pallas as pl
from jax.experimental.pallas import tpu as pltpu
from jax.experimental.pallas import tpu_sc as plsc

B = 16384
NF = 64
D_IMG = 512
NU = 1000000
NC = 2
NS = 16
NW = NC * NS
EPW = B // NW
UPW = 31232
CW = 1280
NCH = 24
TAIL_LO = NW * UPW
OUT_ROWS = B + NW

_sc_mesh = plsc.VectorSubcoreMesh(core_axis_name="c", subcore_axis_name="s")


@functools.partial(
    pl.kernel,
    mesh=_sc_mesh,
    out_type=[
        jax.ShapeDtypeStruct((OUT_ROWS, 128), jnp.float32),
        jax.ShapeDtypeStruct((B,), jnp.float32),
    ],
    scratch_types=[
        pltpu.VMEM((EPW,), jnp.int32),
        pltpu.VMEM((EPW,), jnp.int32),
        pltpu.VMEM((EPW,), jnp.float32),
        pltpu.VMEM((EPW,), jnp.float32),
        pltpu.VMEM((EPW,), jnp.float32),
        pltpu.VMEM((1024,), jnp.int32),
        pltpu.VMEM((B,), jnp.int32),
        pltpu.VMEM((B,), jnp.int32),
        pltpu.VMEM((NF, CW), jnp.float32),
        pltpu.VMEM((NF, 64), jnp.float32),
        pltpu.VMEM((16, 128), jnp.float32),
        pltpu.VMEM((32,), jnp.int32),
        pltpu.VMEM((32,), jnp.int32),
        pltpu.SemaphoreType.DMA,
        pltpu.SemaphoreType.DMA,
    ],
    compiler_params=pltpu.CompilerParams(needs_layout_passes=False),
)
def _sc_gather(user_hbm, item_hbm, ufT_hbm, tailT_hbm, ub_hbm, ib_hbm,
               out_hbm, bias_hbm,
               uidx_v, iidx_v, ubg_v, ibg_v, bias_v, uv_buf,
               el_v, ul_v, strip_v, tail_v, pack_v, tu_v, te_v, sem, sem2):
    wid = lax.axis_index("s") * NC + lax.axis_index("c")
    ebase = wid * EPW

    pltpu.sync_copy(user_hbm.at[pl.ds(ebase, EPW)], uidx_v)
    pltpu.sync_copy(item_hbm.at[pl.ds(ebase, EPW)], iidx_v)
    copies = []
    for j in range(EPW // 128):
        sl = pl.ds(j * 128, 128)
        copies.append(pltpu.async_copy(ub_hbm.at[0].at[uidx_v.at[sl]],
                                       ubg_v.at[sl], sem))
        copies.append(pltpu.async_copy(ib_hbm.at[0].at[iidx_v.at[sl]],
                                       ibg_v.at[sl], sem))
    for c in copies:
        c.wait()
    for j in range(EPW // 16):
        sl = pl.ds(j * 16, 16)
        bias_v[sl] = ubg_v[sl] + ibg_v[sl]
    pltpu.sync_copy(bias_v, bias_hbm.at[pl.ds(ebase, EPW)])

    lo = wid * UPW
    hi = lo + UPW
    is_w0 = wid == 0
    iota16 = lax.iota(jnp.int32, 16)

    def scan_tile(t, cnt):
        def scan_blk(b16, cnt):
            u16 = uv_buf[pl.ds(b16 * 16, 16)]
            m = jnp.logical_and(u16 >= lo, u16 < hi)
            m = jnp.logical_or(m, jnp.logical_and(is_w0, u16 >= TAIL_LO))
            plsc.store_compressed(el_v.at[pl.ds(cnt, 16)],
                                  t * 1024 + b16 * 16 + iota16, mask=m)
            plsc.store_compressed(ul_v.at[pl.ds(cnt, 16)], u16, mask=m)
            return cnt + jnp.sum(m.astype(jnp.int32))
        return lax.fori_loop(0, 64, scan_blk, cnt)

    def bin_tile(t, cnt):
        pltpu.sync_copy(user_hbm.at[pl.ds(t * 1024, 1024)], uv_buf)
        return scan_tile(t, cnt)

    cnt = lax.fori_loop(0, 16, bin_tile, jnp.int32(0))
    nblk = (cnt + 15) >> 4
    dummy = B + wid

    def flush(buf, clo, width, nvalid):
        uu = tu_v[pl.ds(0, 16)]
        ee = te_v[pl.ds(0, 16)]
        lanes = jnp.clip(uu - clo, 0, width - 1)
        mk = iota16 < nvalid
        for k in range(NF):
            kv = jnp.full((16,), k, jnp.int32)
            v = plsc.load_gather(buf, [kv, lanes])
            plsc.store_scatter(pack_v, [iota16, kv], v)
        rows = jnp.where(mk, ee, dummy)
        pltpu.async_copy(pack_v, out_hbm.at[rows], sem2).wait()

    def extract_blocks(buf, clo, width):
        def blk_body(j, pc):
            u16 = ul_v[pl.ds(j * 16, 16)]
            e16 = el_v[pl.ds(j * 16, 16)]
            valid = iota16 < (cnt - j * 16)
            m = jnp.logical_and(valid,
                                jnp.logical_and(u16 >= clo, u16 < clo + width))
            c16 = jnp.sum(m.astype(jnp.int32))

            @pl.when(c16 > 0)
            def _():
                plsc.store_compressed(tu_v.at[pl.ds(pc, 16)], u16, mask=m)
                plsc.store_compressed(te_v.at[pl.ds(pc, 16)], e16, mask=m)

            pc2 = pc + c16

            @pl.when(pc2 >= 16)
            def _():
                flush(buf, clo, width, jnp.int32(16))
                tu_v[pl.ds(0, 16)] = tu_v[pl.ds(16, 16)]
                te_v[pl.ds(0, 16)] = te_v[pl.ds(16, 16)]

            return jnp.where(pc2 >= 16, pc2 - 16, pc2)

        pc = lax.fori_loop(0, nblk, blk_body, jnp.int32(0))

        @pl.when(pc > 0)
        def _():
            flush(buf, clo, width, pc)

    def process_chunk(clo, width):
        strips = [
            pltpu.async_copy(
                ufT_hbm.at[pl.ds(kg * 8, 8), pl.ds(clo, width)],
                strip_v.at[pl.ds(kg * 8, 8), pl.ds(0, width)], sem2)
            for kg in range(8)
        ]
        for s in strips:
            s.wait()
        extract_blocks(strip_v, clo, width)

    @pl.loop(0, NCH)
    def chunk_body(ch):
        process_chunk(lo + ch * CW, CW)

    process_chunk(lo + NCH * CW, 512)

    @pl.when(is_w0)
    def _tail():
        process_chunk(jnp.int32(TAIL_LO), 512)
        pltpu.sync_copy(tailT_hbm, tail_v)
        extract_blocks(tail_v, jnp.int32(TAIL_LO + 512), 64)


BLK = 1024

_tc_matmul = pl.pallas_call(
    lambda img_ref, w_ref, b_ref, o_ref: o_ref.__setitem__(
        ..., jnp.dot(img_ref[...], w_ref[...],
                     preferred_element_type=jnp.float32) + b_ref[...]),
    grid=(B // BLK,),
    in_specs=[
        pl.BlockSpec((BLK, D_IMG), lambda i: (i, 0)),
        pl.BlockSpec((D_IMG, NF), lambda i: (0, 0)),
        pl.BlockSpec((1, NF), lambda i: (0, 0)),
    ],
    out_specs=pl.BlockSpec((BLK, NF), lambda i: (i, 0)),
    out_shape=jax.ShapeDtypeStruct((B, NF), jnp.float32),
)


def _tc_comb_body(imf_ref, pk_ref, bias_ref, out_ref):
    ue = pk_ref[:, :NF]
    out_ref[...] = jnp.sum(imf_ref[...] * ue, axis=1) + bias_ref[...]


_tc_combine = pl.pallas_call(
    _tc_comb_body,
    grid=(B // BLK,),
    in_specs=[
        pl.BlockSpec((BLK, NF), lambda i: (i, 0)),
        pl.BlockSpec((BLK, 128), lambda i: (i, 0)),
        pl.BlockSpec((BLK,), lambda i: (i,)),
    ],
    out_specs=pl.BlockSpec((BLK,), lambda i: (i,)),
    out_shape=jax.ShapeDtypeStruct((B,), jnp.float32),
)


def kernel(image, user, item, user_factors, user_biases, item_biases,
           W_img, b_img):
    user = user.astype(jnp.int32)
    item = item.astype(jnp.int32)
    ufT = user_factors.T
    tailT = ufT[:, TAIL_LO + 512:]
    ubT = user_biases.T
    ibT = item_biases.T
    packed, bias = _sc_gather(user, item, ufT, tailT, ubT, ibT)
    imf = _tc_matmul(image, W_img, b_img.reshape(1, NF))
    return _tc_combine(imf, packed, bias)


# speedup vs baseline: 37.3338x; 1.1624x over previous
import functools

import jax
import jax.numpy as jnp
from jax import lax
from jax.experimental import pallas as pl
from jax.experimental.pallas import tpu as pltpu
from jax.experimental.pallas import tpu_sc as plsc

B = 16384
NF = 64
D_IMG = 512
NU = 1000000
NC = 2
NS = 16
NW = NC * NS
EPW = B // NW
UPW = 31232
CW = 640
NCH = 48
TAIL_LO = NW * UPW
OUT_ROWS = B + NW

_sc_mesh = plsc.VectorSubcoreMesh(core_axis_name="c", subcore_axis_name="s")


@functools.partial(
    pl.kernel,
    mesh=_sc_mesh,
    out_type=[
        jax.ShapeDtypeStruct((OUT_ROWS, 128), jnp.float32),
        jax.ShapeDtypeStruct((B,), jnp.float32),
    ],
    scratch_types=[
        pltpu.VMEM((EPW,), jnp.int32),
        pltpu.VMEM((EPW,), jnp.int32),
        pltpu.VMEM((EPW,), jnp.float32),
        pltpu.VMEM((EPW,), jnp.float32),
        pltpu.VMEM((EPW,), jnp.float32),
        pltpu.VMEM((1024,), jnp.int32),
        pltpu.VMEM((B,), jnp.int32),
        pltpu.VMEM((B,), jnp.int32),
        pltpu.VMEM((NF, CW), jnp.float32),
        pltpu.VMEM((NF, CW), jnp.float32),
        pltpu.VMEM((NF, 64), jnp.float32),
        pltpu.VMEM((16, 128), jnp.float32),
        pltpu.VMEM((32,), jnp.int32),
        pltpu.VMEM((32,), jnp.int32),
        pltpu.SemaphoreType.DMA,
        pltpu.SemaphoreType.DMA,
        pltpu.SemaphoreType.DMA,
        pltpu.SemaphoreType.DMA,
    ],
    compiler_params=pltpu.CompilerParams(needs_layout_passes=False),
)
def _sc_gather(user_hbm, item_hbm, ufT_hbm, tailT_hbm, ub_hbm, ib_hbm,
               out_hbm, bias_hbm,
               uidx_v, iidx_v, ubg_v, ibg_v, bias_v, uv_buf,
               el_v, ul_v, strip0_v, strip1_v, tail_v,
               pack_v, tu_v, te_v, sem, sem2, semA, semB):
    wid = lax.axis_index("s") * NC + lax.axis_index("c")
    ebase = wid * EPW
    lo = wid * UPW
    hi = lo + UPW
    is_w0 = wid == 0
    iota16 = lax.iota(jnp.int32, 16)

    pltpu.sync_copy(user_hbm.at[pl.ds(ebase, EPW)], uidx_v)
    pltpu.sync_copy(item_hbm.at[pl.ds(ebase, EPW)], iidx_v)
    bias_copies = []
    for j in range(EPW // 128):
        sl = pl.ds(j * 128, 128)
        bias_copies.append(pltpu.async_copy(ub_hbm.at[0].at[uidx_v.at[sl]],
                                            ubg_v.at[sl], sem))
        bias_copies.append(pltpu.async_copy(ib_hbm.at[0].at[iidx_v.at[sl]],
                                            ibg_v.at[sl], sem))

    def fire_strips(c, buf, dsem):
        return [
            pltpu.async_copy(
                ufT_hbm.at[pl.ds(kg * 8, 8), pl.ds(lo + c * CW, CW)],
                buf.at[pl.ds(kg * 8, 8)], dsem)
            for kg in range(8)
        ]

    def drain_strips(c, buf, dsem):
        for kg in range(8):
            pltpu.make_async_copy(
                ufT_hbm.at[pl.ds(kg * 8, 8), pl.ds(lo + c * CW, CW)],
                buf.at[pl.ds(kg * 8, 8)], dsem).wait()

    fire_strips(0, strip0_v, semA)
    fire_strips(1, strip1_v, semB)

    def scan_tile(t, cnt):
        def scan_blk(b16, cnt):
            u16 = uv_buf[pl.ds(b16 * 16, 16)]
            m = jnp.logical_and(u16 >= lo, u16 < hi)
            m = jnp.logical_or(m, jnp.logical_and(is_w0, u16 >= TAIL_LO))
            plsc.store_compressed(el_v.at[pl.ds(cnt, 16)],
                                  t * 1024 + b16 * 16 + iota16, mask=m)
            plsc.store_compressed(ul_v.at[pl.ds(cnt, 16)], u16, mask=m)
            return cnt + jnp.sum(m.astype(jnp.int32))
        return lax.fori_loop(0, 64, scan_blk, cnt)

    def bin_tile(t, cnt):
        pltpu.sync_copy(user_hbm.at[pl.ds(t * 1024, 1024)], uv_buf)
        return scan_tile(t, cnt)

    cnt = lax.fori_loop(0, 16, bin_tile, jnp.int32(0))
    nblk = (cnt + 15) >> 4
    dummy = B + wid

    for c in bias_copies:
        c.wait()
    for j in range(EPW // 16):
        sl = pl.ds(j * 16, 16)
        bias_v[sl] = ubg_v[sl] + ibg_v[sl]
    pltpu.sync_copy(bias_v, bias_hbm.at[pl.ds(ebase, EPW)])

    def flush(buf, clo, width, nvalid):
        uu = tu_v[pl.ds(0, 16)]
        ee = te_v[pl.ds(0, 16)]
        lanes = jnp.clip(uu - clo, 0, width - 1)
        mk = iota16 < nvalid
        for k in range(NF):
            kv = jnp.full((16,), k, jnp.int32)
            v = plsc.load_gather(buf, [kv, lanes])
            plsc.store_scatter(pack_v, [iota16, kv], v)
        rows = jnp.where(mk, ee, dummy)
        pltpu.async_copy(pack_v, out_hbm.at[rows], sem2).wait()

    def extract_blocks(buf, clo, width):
        def blk_body(j, pc):
            u16 = ul_v[pl.ds(j * 16, 16)]
            e16 = el_v[pl.ds(j * 16, 16)]
            valid = iota16 < (cnt - j * 16)
            m = jnp.logical_and(valid,
                                jnp.logical_and(u16 >= clo, u16 < clo + width))
            c16 = jnp.sum(m.astype(jnp.int32))

            @pl.when(c16 > 0)
            def _():
                plsc.store_compressed(tu_v.at[pl.ds(pc, 16)], u16, mask=m)
                plsc.store_compressed(te_v.at[pl.ds(pc, 16)], e16, mask=m)

            pc2 = pc + c16

            @pl.when(pc2 >= 16)
            def _():
                flush(buf, clo, width, jnp.int32(16))
                tu_v[pl.ds(0, 16)] = tu_v[pl.ds(16, 16)]
                te_v[pl.ds(0, 16)] = te_v[pl.ds(16, 16)]

            return jnp.where(pc2 >= 16, pc2 - 16, pc2)

        pc = lax.fori_loop(0, nblk, blk_body, jnp.int32(0))

        @pl.when(pc > 0)
        def _():
            flush(buf, clo, width, pc)

    @pl.loop(0, NCH // 2)
    def chunk_pair(it):
        for slot, buf, dsem in ((0, strip0_v, semA), (1, strip1_v, semB)):
            c = it * 2 + slot
            drain_strips(c, buf, dsem)
            extract_blocks(buf, lo + c * CW, CW)

            @pl.when(c + 2 < NCH)
            def _():
                fire_strips(c + 2, buf, dsem)

    def process_512(clo):
        strips = [
            pltpu.async_copy(
                ufT_hbm.at[pl.ds(kg * 8, 8), pl.ds(clo, 512)],
                strip0_v.at[pl.ds(kg * 8, 8), pl.ds(0, 512)], sem2)
            for kg in range(8)
        ]
        for s in strips:
            s.wait()
        extract_blocks(strip0_v, clo, 512)

    process_512(lo + NCH * CW)

    @pl.when(is_w0)
    def _tail():
        process_512(jnp.int32(TAIL_LO))
        pltpu.sync_copy(tailT_hbm, tail_v)
        extract_blocks(tail_v, jnp.int32(TAIL_LO + 512), 64)


BLK = 1024

_tc_matmul = pl.pallas_call(
    lambda img_ref, w_ref, b_ref, o_ref: o_ref.__setitem__(
        ..., jnp.dot(img_ref[...], w_ref[...],
                     preferred_element_type=jnp.float32) + b_ref[...]),
    grid=(B // BLK,),
    in_specs=[
        pl.BlockSpec((BLK, D_IMG), lambda i: (i, 0)),
        pl.BlockSpec((D_IMG, NF), lambda i: (0, 0)),
        pl.BlockSpec((1, NF), lambda i: (0, 0)),
    ],
    out_specs=pl.BlockSpec((BLK, NF), lambda i: (i, 0)),
    out_shape=jax.ShapeDtypeStruct((B, NF), jnp.float32),
)


def _tc_comb_body(imf_ref, pk_ref, bias_ref, out_ref):
    ue = pk_ref[:, :NF]
    out_ref[...] = jnp.sum(imf_ref[...] * ue, axis=1) + bias_ref[...]


_tc_combine = pl.pallas_call(
    _tc_comb_body,
    grid=(B // BLK,),
    in_specs=[
        pl.BlockSpec((BLK, NF), lambda i: (i, 0)),
        pl.BlockSpec((BLK, 128), lambda i: (i, 0)),
        pl.BlockSpec((BLK,), lambda i: (i,)),
    ],
    out_specs=pl.BlockSpec((BLK,), lambda i: (i,)),
    out_shape=jax.ShapeDtypeStruct((B,), jnp.float32),
)


def kernel(image, user, item, user_factors, user_biases, item_biases,
           W_img, b_img):
    user = user.astype(jnp.int32)
    item = item.astype(jnp.int32)
    ufT = user_factors.T
    tailT = ufT[:, TAIL_LO + 512:]
    ubT = user_biases.T
    ibT = item_biases.T
    packed, bias = _sc_gather(user, item, ufT, tailT, ubT, ibT)
    imf = _tc_matmul(image, W_img, b_img.reshape(1, NF))
    return _tc_combine(imf, packed, bias)
